# baseline (speedup 1.0000x reference)
import functools

import jax
import jax.numpy as jnp
from jax import lax
from jax.experimental import pallas as pl
from jax.experimental.pallas import tpu as pltpu
from jax.experimental.pallas import tpu_sc as plsc

N = 10000
E = 160000
IN_C = 128
OUT_H = 4
T = 16
GW = 128
CW = 16
NC = 2
NS = 16
NW = NC * NS
EPW = E // NW
CH = 40
CHR = CH // 8
NCHUNK = EPW // CH
NACC = 10240
RPT = NACC // NS
ZR = 40
BN = 2000
BE = 16000


def _tc_prep_body(x_ref, gn_ref, bn_ref, gr_ref, br_ref, rn_ref, rr_ref,
                  bias_ref, gcat_ref, c16_ref, base_ref):
    x = x_ref[...]
    out_parts = []
    base_parts = []
    for g_ref, b_ref, rhs_ref in ((gn_ref, bn_ref, rn_ref),
                                  (gr_ref, br_ref, rr_ref)):
        mu = jnp.mean(x, axis=-1, keepdims=True)
        var = jnp.mean((x - mu) ** 2, axis=-1, keepdims=True)
        h = (x - mu) / jnp.sqrt(var + 1e-5) * g_ref[...] + b_ref[...]
        h = jax.nn.relu(h)
        p = jnp.dot(h, rhs_ref[...], precision="highest",
                    preferred_element_type=jnp.float32)
        out_parts.append(p)
    pn, pr = out_parts
    zeros = jnp.zeros((x.shape[0], 8), jnp.float32)
    gcat_ref[...] = jnp.concatenate([pn[:, :64], pr[:, :64]], axis=-1)
    c16_ref[...] = jnp.concatenate(
        [pn[:, 64:68], pr[:, 64:68], zeros], axis=-1)
    base_ref[...] = jnp.concatenate(
        [pn[:, 68:72], pr[:, 68:72]], axis=-1) + bias_ref[...]


def _tc_repack_body(ea_ref, out_ref):
    x = ea_ref[...].reshape(BE // 8, 8, T)
    parts = [x[:, s, :] for s in range(8)]
    out_ref[...] = jnp.concatenate(parts, axis=1)


def _sc_edge_body(gcat_hbm, c16_hbm, ea_hbm, src_hbm, dst_hbm,
                  outg_hbm, outc_hbm,
                  qshg, qshc, srcall, dstall, av, arv, rows, crows,
                  zbuf, zbufc, sem0, sem1):
    cid = lax.axis_index("c")
    sid = lax.axis_index("s")
    wid = sid * NC + cid
    sems = (sem0, sem1)

    z16 = jnp.zeros((16,), jnp.float32)

    def zfill(i, _):
        for c in range(GW // 16):
            zbuf[i, pl.ds(16 * c, 16)] = z16
        zbufc[i, :] = z16
        return 0

    lax.fori_loop(0, ZR, zfill, 0)
    for j in range(RPT // ZR):
        pltpu.sync_copy(zbuf, qshg.at[pl.ds(sid * RPT + j * ZR, ZR)])
        pltpu.sync_copy(zbufc, qshc.at[pl.ds(sid * RPT + j * ZR, ZR)])
    plsc.subcore_barrier()

    pltpu.sync_copy(src_hbm.at[wid], srcall)
    pltpu.sync_copy(dst_hbm.at[wid], dstall)

    def issue(k, b):
        rbase = (wid * EPW) // 8 + k * CHR
        pltpu.async_copy(ea_hbm.at[pl.ds(rbase, CHR)], av.at[b], sems[b])
        pltpu.async_copy(ea_hbm.at[pl.ds(E // 8 - rbase - CHR, CHR)],
                         arv.at[b], sems[b])
        pltpu.async_copy(gcat_hbm.at[srcall.at[k]], rows.at[b], sems[b])
        pltpu.async_copy(c16_hbm.at[srcall.at[k]], crows.at[b], sems[b])

    def drain(b):
        pltpu.make_async_copy(ea_hbm.at[pl.ds(0, CHR)], av.at[b],
                              sems[b]).wait()
        pltpu.make_async_copy(ea_hbm.at[pl.ds(0, CHR)], arv.at[b],
                              sems[b]).wait()
        pltpu.make_async_copy(gcat_hbm.at[pl.ds(0, CH)], rows.at[b],
                              sems[b]).wait()
        pltpu.make_async_copy(c16_hbm.at[pl.ds(0, CH)], crows.at[b],
                              sems[b]).wait()

    def process(k, b):
        drain(b)
        rb = rows.at[b]
        avb = av.at[b]
        arvb = arv.at[b]

        def edge8(rr, _):
            for s in range(8):
                i = rr * 8 + s
                an = avb[rr, pl.ds(16 * s, 16)]
                ar = arvb[CHR - 1 - rr, pl.ds(16 * (7 - s), 16)]
                for c in range(4):
                    rb[i, pl.ds(16 * c, 16)] = (
                        rb[i, pl.ds(16 * c, 16)] * an)
                for c in range(4, 8):
                    rb[i, pl.ds(16 * c, 16)] = (
                        rb[i, pl.ds(16 * c, 16)] * ar)
            return 0

        lax.fori_loop(0, CHR, edge8, 0)
        pltpu.sync_copy(rb, qshg.at[dstall.at[k]], add=True)
        pltpu.sync_copy(crows.at[b], qshc.at[dstall.at[k]], add=True)

    issue(0, 0)

    def outer(g, _):
        for b in range(2):
            k = 2 * g + b
            issue(k + 1, 1 - b)
            process(k, b)
        return 0

    lax.fori_loop(0, NCHUNK // 2, outer, 0)
    if NCHUNK % 2 == 1:
        process(NCHUNK - 1, (NCHUNK - 1) % 2)
    plsc.subcore_barrier()
    pltpu.sync_copy(qshg.at[pl.ds(sid * RPT, RPT)],
                    outg_hbm.at[cid, pl.ds(sid * RPT, RPT)])
    pltpu.sync_copy(qshc.at[pl.ds(sid * RPT, RPT)],
                    outc_hbm.at[cid, pl.ds(sid * RPT, RPT)])


def _tc_finish_body(pg_ref, pc_ref, base_ref, out_ref):
    k = lax.broadcasted_iota(jnp.int32, (GW, 8), 0)
    c = lax.broadcasted_iota(jnp.int32, (GW, 8), 1)
    s = jnp.where(k // 16 == c, 1.0, 0.0).astype(jnp.float32)
    pg = pg_ref[0] + pg_ref[1]
    pc = pc_ref[0] + pc_ref[1]
    out_ref[...] = base_ref[...] + pc[:, :8] + jnp.dot(
        pg, s, precision="highest", preferred_element_type=jnp.float32)


def kernel(x, edge_index, edge_attr, ln_g_n, ln_b_n, ew_n, eb_n, root_n, b_n,
           ln_g_r, ln_b_r, ew_r, eb_r, root_r, b_r):
    def rhs(ew, eb, root):
        w = ew.reshape(IN_C, OUT_H, T).reshape(IN_C, OUT_H * T)
        return jnp.concatenate([w, eb.reshape(IN_C, OUT_H), root], axis=1)

    rhs_n = rhs(ew_n, eb_n, root_n)
    rhs_r = rhs(ew_r, eb_r, root_r)
    bias = jnp.concatenate([b_n, b_r]).reshape(1, 8)

    grid = N // BN
    full = lambda s: pl.BlockSpec(s, lambda j: (0,) * len(s))
    gcat, c16, base = pl.pallas_call(
        _tc_prep_body,
        grid=(grid,),
        in_specs=[
            pl.BlockSpec((BN, IN_C), lambda j: (j, 0)),
            full((1, IN_C)), full((1, IN_C)),
            full((1, IN_C)), full((1, IN_C)),
            full((IN_C, 72)), full((IN_C, 72)),
            full((1, 8)),
        ],
        out_specs=[
            pl.BlockSpec((BN, GW), lambda j: (j, 0)),
            pl.BlockSpec((BN, CW), lambda j: (j, 0)),
            pl.BlockSpec((BN, 8), lambda j: (j, 0)),
        ],
        out_shape=[
            jax.ShapeDtypeStruct((N, GW), jnp.float32),
            jax.ShapeDtypeStruct((N, CW), jnp.float32),
            jax.ShapeDtypeStruct((N, 8), jnp.float32),
        ],
    )(x, ln_g_n.reshape(1, IN_C), ln_b_n.reshape(1, IN_C),
      ln_g_r.reshape(1, IN_C), ln_b_r.reshape(1, IN_C),
      rhs_n, rhs_r, bias)

    ea128 = pl.pallas_call(
        _tc_repack_body,
        grid=(E // BE,),
        in_specs=[pl.BlockSpec((BE, T), lambda j: (j, 0))],
        out_specs=pl.BlockSpec((BE // 8, GW), lambda j: (j, 0)),
        out_shape=jax.ShapeDtypeStruct((E // 8, GW), jnp.float32),
    )(edge_attr)

    mesh = plsc.VectorSubcoreMesh(core_axis_name="c", subcore_axis_name="s",
                                  num_cores=NC, num_subcores=NS)
    partg, partc = pl.kernel(
        _sc_edge_body,
        out_type=[
            jax.ShapeDtypeStruct((NC, NACC, GW), jnp.float32),
            jax.ShapeDtypeStruct((NC, NACC, CW), jnp.float32),
        ],
        mesh=mesh,
        compiler_params=pltpu.CompilerParams(use_tc_tiling_on_sc=False),
        scratch_types=[
            pltpu.VMEM_SHARED((NACC, GW), jnp.float32),
            pltpu.VMEM_SHARED((NACC, CW), jnp.float32),
            pltpu.VMEM((NCHUNK, CH), jnp.int32),
            pltpu.VMEM((NCHUNK, CH), jnp.int32),
            pltpu.VMEM((2, CHR, GW), jnp.float32),
            pltpu.VMEM((2, CHR, GW), jnp.float32),
            pltpu.VMEM((2, CH, GW), jnp.float32),
            pltpu.VMEM((2, CH, CW), jnp.float32),
            pltpu.VMEM((ZR, GW), jnp.float32),
            pltpu.VMEM((ZR, CW), jnp.float32),
            pltpu.SemaphoreType.DMA,
            pltpu.SemaphoreType.DMA,
        ],
    )(gcat, c16, ea128,
      edge_index[0].reshape(NW, NCHUNK, CH),
      edge_index[1].reshape(NW, NCHUNK, CH))

    out = pl.pallas_call(
        _tc_finish_body,
        grid=(grid,),
        in_specs=[
            pl.BlockSpec((NC, BN, GW), lambda j: (0, j, 0)),
            pl.BlockSpec((NC, BN, CW), lambda j: (0, j, 0)),
            pl.BlockSpec((BN, 8), lambda j: (j, 0)),
        ],
        out_specs=pl.BlockSpec((BN, 8), lambda j: (j, 0)),
        out_shape=jax.ShapeDtypeStruct((N, 8), jnp.float32),
    )(partg, partc, base)
    return out


# speedup vs baseline: 1.1671x; 1.1671x over previous
import functools

import jax
import jax.numpy as jnp
from jax import lax
from jax.experimental import pallas as pl
from jax.experimental.pallas import tpu as pltpu
from jax.experimental.pallas import tpu_sc as plsc

N = 10000
E = 160000
IN_C = 128
OUT_H = 4
T = 16
GW = 128
CW = 16
NC = 2
NS = 16
NW = NC * NS
EPW = E // NW
CH = 50
NCHUNK = EPW // CH
NACC = 10240
RPT = NACC // NS
ZR = 40
BN = 2000


def _tc_prep_body(x_ref, gn_ref, bn_ref, gr_ref, br_ref, rn_ref, rr_ref,
                  bias_ref, gcat_ref, c16_ref, base_ref):
    x = x_ref[...]
    out_parts = []
    base_parts = []
    for g_ref, b_ref, rhs_ref in ((gn_ref, bn_ref, rn_ref),
                                  (gr_ref, br_ref, rr_ref)):
        mu = jnp.mean(x, axis=-1, keepdims=True)
        var = jnp.mean((x - mu) ** 2, axis=-1, keepdims=True)
        h = (x - mu) / jnp.sqrt(var + 1e-5) * g_ref[...] + b_ref[...]
        h = jax.nn.relu(h)
        p = jnp.dot(h, rhs_ref[...], precision="highest",
                    preferred_element_type=jnp.float32)
        out_parts.append(p)
    pn, pr = out_parts
    zeros = jnp.zeros((x.shape[0], 8), jnp.float32)
    gcat_ref[...] = jnp.concatenate([pn[:, :64], pr[:, :64]], axis=-1)
    c16_ref[...] = jnp.concatenate(
        [pn[:, 64:68], pr[:, 64:68], zeros], axis=-1)
    base_ref[...] = jnp.concatenate(
        [pn[:, 68:72], pr[:, 68:72]], axis=-1) + bias_ref[...]


def _sc_edge_body(gcat_hbm, c16_hbm, ea_hbm, src_hbm, dst_hbm,
                  outg_hbm, outc_hbm,
                  qshg, qshc, srcall, dstall, av, arv, rows, crows,
                  zbuf, zbufc, sem0, sem1):
    cid = lax.axis_index("c")
    sid = lax.axis_index("s")
    wid = sid * NC + cid
    sems = (sem0, sem1)

    z16 = jnp.zeros((16,), jnp.float32)

    def zfill(i, _):
        for c in range(GW // 16):
            zbuf[i, pl.ds(16 * c, 16)] = z16
        zbufc[i, :] = z16
        return 0

    lax.fori_loop(0, ZR, zfill, 0)
    for j in range(RPT // ZR):
        pltpu.sync_copy(zbuf, qshg.at[pl.ds(sid * RPT + j * ZR, ZR)])
        pltpu.sync_copy(zbufc, qshc.at[pl.ds(sid * RPT + j * ZR, ZR)])
    plsc.subcore_barrier()

    pltpu.sync_copy(src_hbm.at[wid], srcall)
    pltpu.sync_copy(dst_hbm.at[wid], dstall)

    def issue(k, b):
        base = wid * EPW + k * CH
        pltpu.async_copy(ea_hbm.at[pl.ds(base, CH)], av.at[b], sems[b])
        pltpu.async_copy(ea_hbm.at[pl.ds(E - base - CH, CH)], arv.at[b],
                         sems[b])
        pltpu.async_copy(gcat_hbm.at[srcall.at[k]], rows.at[b], sems[b])
        pltpu.async_copy(c16_hbm.at[srcall.at[k]], crows.at[b], sems[b])

    def drain(b):
        pltpu.make_async_copy(ea_hbm.at[pl.ds(0, CH)], av.at[b],
                              sems[b]).wait()
        pltpu.make_async_copy(ea_hbm.at[pl.ds(0, CH)], arv.at[b],
                              sems[b]).wait()
        pltpu.make_async_copy(gcat_hbm.at[pl.ds(0, CH)], rows.at[b],
                              sems[b]).wait()
        pltpu.make_async_copy(c16_hbm.at[pl.ds(0, CH)], crows.at[b],
                              sems[b]).wait()

    issue(0, 0)

    def outer(g, _):
        for b in range(2):
            k = 2 * g + b
            nb = 1 - b

            @pl.when(k + 1 < NCHUNK)
            def _():
                issue(k + 1, nb)

            drain(b)
            rb = rows.at[b]
            avb = av.at[b]
            arvb = arv.at[b]

            def edge(i, _):
                an = avb[i, :]
                ar = arvb[CH - 1 - i, :]
                for c in range(4):
                    rb[i, pl.ds(16 * c, 16)] = rb[i, pl.ds(16 * c, 16)] * an
                for c in range(4, 8):
                    rb[i, pl.ds(16 * c, 16)] = rb[i, pl.ds(16 * c, 16)] * ar
                return 0

            lax.fori_loop(0, CH, edge, 0)
            pltpu.sync_copy(rb, qshg.at[dstall.at[k]], add=True)
            pltpu.sync_copy(crows.at[b], qshc.at[dstall.at[k]], add=True)
        return 0

    lax.fori_loop(0, NCHUNK // 2, outer, 0)
    plsc.subcore_barrier()
    pltpu.sync_copy(qshg.at[pl.ds(sid * RPT, RPT)],
                    outg_hbm.at[cid, pl.ds(sid * RPT, RPT)])
    pltpu.sync_copy(qshc.at[pl.ds(sid * RPT, RPT)],
                    outc_hbm.at[cid, pl.ds(sid * RPT, RPT)])


def _tc_finish_body(pg_ref, pc_ref, base_ref, out_ref):
    k = lax.broadcasted_iota(jnp.int32, (GW, 8), 0)
    c = lax.broadcasted_iota(jnp.int32, (GW, 8), 1)
    s = jnp.where(k // 16 == c, 1.0, 0.0).astype(jnp.float32)
    pg = pg_ref[0] + pg_ref[1]
    pc = pc_ref[0] + pc_ref[1]
    out_ref[...] = base_ref[...] + pc[:, :8] + jnp.dot(
        pg, s, precision="highest", preferred_element_type=jnp.float32)


def kernel(x, edge_index, edge_attr, ln_g_n, ln_b_n, ew_n, eb_n, root_n, b_n,
           ln_g_r, ln_b_r, ew_r, eb_r, root_r, b_r):
    def rhs(ew, eb, root):
        w = ew.reshape(IN_C, OUT_H, T).reshape(IN_C, OUT_H * T)
        return jnp.concatenate([w, eb.reshape(IN_C, OUT_H), root], axis=1)

    rhs_n = rhs(ew_n, eb_n, root_n)
    rhs_r = rhs(ew_r, eb_r, root_r)
    bias = jnp.concatenate([b_n, b_r]).reshape(1, 8)

    grid = N // BN
    full = lambda s: pl.BlockSpec(s, lambda j: (0,) * len(s))
    gcat, c16, base = pl.pallas_call(
        _tc_prep_body,
        grid=(grid,),
        in_specs=[
            pl.BlockSpec((BN, IN_C), lambda j: (j, 0)),
            full((1, IN_C)), full((1, IN_C)),
            full((1, IN_C)), full((1, IN_C)),
            full((IN_C, 72)), full((IN_C, 72)),
            full((1, 8)),
        ],
        out_specs=[
            pl.BlockSpec((BN, GW), lambda j: (j, 0)),
            pl.BlockSpec((BN, CW), lambda j: (j, 0)),
            pl.BlockSpec((BN, 8), lambda j: (j, 0)),
        ],
        out_shape=[
            jax.ShapeDtypeStruct((N, GW), jnp.float32),
            jax.ShapeDtypeStruct((N, CW), jnp.float32),
            jax.ShapeDtypeStruct((N, 8), jnp.float32),
        ],
    )(x, ln_g_n.reshape(1, IN_C), ln_b_n.reshape(1, IN_C),
      ln_g_r.reshape(1, IN_C), ln_b_r.reshape(1, IN_C),
      rhs_n, rhs_r, bias)

    mesh = plsc.VectorSubcoreMesh(core_axis_name="c", subcore_axis_name="s",
                                  num_cores=NC, num_subcores=NS)
    partg, partc = pl.kernel(
        _sc_edge_body,
        out_type=[
            jax.ShapeDtypeStruct((NC, NACC, GW), jnp.float32),
            jax.ShapeDtypeStruct((NC, NACC, CW), jnp.float32),
        ],
        mesh=mesh,
        compiler_params=pltpu.CompilerParams(use_tc_tiling_on_sc=False),
        scratch_types=[
            pltpu.VMEM_SHARED((NACC, GW), jnp.float32),
            pltpu.VMEM_SHARED((NACC, CW), jnp.float32),
            pltpu.VMEM((NCHUNK, CH), jnp.int32),
            pltpu.VMEM((NCHUNK, CH), jnp.int32),
            pltpu.VMEM((2, CH, T), jnp.float32),
            pltpu.VMEM((2, CH, T), jnp.float32),
            pltpu.VMEM((2, CH, GW), jnp.float32),
            pltpu.VMEM((2, CH, CW), jnp.float32),
            pltpu.VMEM((ZR, GW), jnp.float32),
            pltpu.VMEM((ZR, CW), jnp.float32),
            pltpu.SemaphoreType.DMA,
            pltpu.SemaphoreType.DMA,
        ],
    )(gcat, c16, edge_attr,
      edge_index[0].reshape(NW, NCHUNK, CH),
      edge_index[1].reshape(NW, NCHUNK, CH))

    out = pl.pallas_call(
        _tc_finish_body,
        grid=(grid,),
        in_specs=[
            pl.BlockSpec((NC, BN, GW), lambda j: (0, j, 0)),
            pl.BlockSpec((NC, BN, CW), lambda j: (0, j, 0)),
            pl.BlockSpec((BN, 8), lambda j: (j, 0)),
        ],
        out_specs=pl.BlockSpec((BN, 8), lambda j: (j, 0)),
        out_shape=jax.ShapeDtypeStruct((N, 8), jnp.float32),
    )(partg, partc, base)
    return out


# speedup vs baseline: 1.1836x; 1.0142x over previous
import functools

import jax
import jax.numpy as jnp
from jax import lax
from jax.experimental import pallas as pl
from jax.experimental.pallas import tpu as pltpu
from jax.experimental.pallas import tpu_sc as plsc

N = 10000
E = 160000
IN_C = 128
OUT_H = 4
T = 16
GW = 128
CW = 16
NC = 2
NS = 16
NW = NC * NS
EPW = E // NW
CH = 50
NCHUNK = EPW // CH
NACC = 10240
RPT = NACC // NS
ZR = 40
BN = 2000


def _tc_prep_body(x_ref, gn_ref, bn_ref, gr_ref, br_ref, rn_ref, rr_ref,
                  bias_ref, gcat_ref, c16_ref, base_ref):
    x = x_ref[...]
    out_parts = []
    base_parts = []
    for g_ref, b_ref, rhs_ref in ((gn_ref, bn_ref, rn_ref),
                                  (gr_ref, br_ref, rr_ref)):
        mu = jnp.mean(x, axis=-1, keepdims=True)
        var = jnp.mean((x - mu) ** 2, axis=-1, keepdims=True)
        h = (x - mu) / jnp.sqrt(var + 1e-5) * g_ref[...] + b_ref[...]
        h = jax.nn.relu(h)
        p = jnp.dot(h, rhs_ref[...], precision="highest",
                    preferred_element_type=jnp.float32)
        out_parts.append(p)
    pn, pr = out_parts
    zeros = jnp.zeros((x.shape[0], 8), jnp.float32)
    gcat_ref[...] = jnp.concatenate([pn[:, :64], pr[:, :64]], axis=-1)
    c16_ref[...] = jnp.concatenate(
        [pn[:, 64:68], pr[:, 64:68], zeros], axis=-1)
    base_ref[...] = jnp.concatenate(
        [pn[:, 68:72], pr[:, 68:72]], axis=-1) + bias_ref[...]


def _sc_edge_body(gcat_hbm, c16_hbm, ea_hbm, src_hbm, dst_hbm,
                  outg_hbm, outc_hbm,
                  qshg, qshc, srcall, dstall, av, arv, rows, crows,
                  zbuf, zbufc, sem0, sem1, ssem0, ssem1):
    cid = lax.axis_index("c")
    sid = lax.axis_index("s")
    wid = sid * NC + cid
    sems = (sem0, sem1)
    ssems = (ssem0, ssem1)

    z16 = jnp.zeros((16,), jnp.float32)

    def zfill(i, _):
        for c in range(GW // 16):
            zbuf[i, pl.ds(16 * c, 16)] = z16
        zbufc[i, :] = z16
        return 0

    lax.fori_loop(0, ZR, zfill, 0)
    for j in range(RPT // ZR):
        pltpu.sync_copy(zbuf, qshg.at[pl.ds(sid * RPT + j * ZR, ZR)])
        pltpu.sync_copy(zbufc, qshc.at[pl.ds(sid * RPT + j * ZR, ZR)])
    plsc.subcore_barrier()

    pltpu.sync_copy(src_hbm.at[wid], srcall)
    pltpu.sync_copy(dst_hbm.at[wid], dstall)

    def issue(k, b):
        base = wid * EPW + k * CH
        pltpu.async_copy(ea_hbm.at[pl.ds(base, CH)], av.at[b], sems[b])
        pltpu.async_copy(ea_hbm.at[pl.ds(E - base - CH, CH)], arv.at[b],
                         sems[b])
        pltpu.async_copy(gcat_hbm.at[srcall.at[k]], rows.at[b], sems[b])
        pltpu.async_copy(c16_hbm.at[srcall.at[k]], crows.at[b], sems[b])

    def drain(b):
        pltpu.make_async_copy(ea_hbm.at[pl.ds(0, CH)], av.at[b],
                              sems[b]).wait()
        pltpu.make_async_copy(ea_hbm.at[pl.ds(0, CH)], arv.at[b],
                              sems[b]).wait()
        pltpu.make_async_copy(gcat_hbm.at[pl.ds(0, CH)], rows.at[b],
                              sems[b]).wait()
        pltpu.make_async_copy(c16_hbm.at[pl.ds(0, CH)], crows.at[b],
                              sems[b]).wait()

    def drain_scatter(b):
        pltpu.make_async_copy(gcat_hbm.at[pl.ds(0, CH)], rows.at[b],
                              ssems[b]).wait()
        pltpu.make_async_copy(c16_hbm.at[pl.ds(0, CH)], crows.at[b],
                              ssems[b]).wait()

    issue(0, 0)

    def outer(g, _):
        for b in range(2):
            k = 2 * g + b
            nb = 1 - b

            @pl.when(k + 1 < NCHUNK)
            def _():
                @pl.when(k >= 1)
                def _():
                    drain_scatter(nb)

                issue(k + 1, nb)

            drain(b)
            rb = rows.at[b]
            avb = av.at[b]
            arvb = arv.at[b]

            def edge(i, _):
                an = avb[i, :]
                ar = arvb[CH - 1 - i, :]
                for c in range(4):
                    rb[i, pl.ds(16 * c, 16)] = rb[i, pl.ds(16 * c, 16)] * an
                for c in range(4, 8):
                    rb[i, pl.ds(16 * c, 16)] = rb[i, pl.ds(16 * c, 16)] * ar
                return 0

            lax.fori_loop(0, CH, edge, 0)
            pltpu.async_copy(rb, qshg.at[dstall.at[k]], ssems[b], add=True)
            pltpu.async_copy(crows.at[b], qshc.at[dstall.at[k]], ssems[b],
                             add=True)
        return 0

    lax.fori_loop(0, NCHUNK // 2, outer, 0)
    drain_scatter(0)
    drain_scatter(1)
    plsc.subcore_barrier()
    pltpu.sync_copy(qshg.at[pl.ds(sid * RPT, RPT)],
                    outg_hbm.at[cid, pl.ds(sid * RPT, RPT)])
    pltpu.sync_copy(qshc.at[pl.ds(sid * RPT, RPT)],
                    outc_hbm.at[cid, pl.ds(sid * RPT, RPT)])


def _tc_finish_body(pg_ref, pc_ref, base_ref, out_ref):
    k = lax.broadcasted_iota(jnp.int32, (GW, 8), 0)
    c = lax.broadcasted_iota(jnp.int32, (GW, 8), 1)
    s = jnp.where(k // 16 == c, 1.0, 0.0).astype(jnp.float32)
    pg = pg_ref[0] + pg_ref[1]
    pc = pc_ref[0] + pc_ref[1]
    out_ref[...] = base_ref[...] + pc[:, :8] + jnp.dot(
        pg, s, precision="highest", preferred_element_type=jnp.float32)


def kernel(x, edge_index, edge_attr, ln_g_n, ln_b_n, ew_n, eb_n, root_n, b_n,
           ln_g_r, ln_b_r, ew_r, eb_r, root_r, b_r):
    def rhs(ew, eb, root):
        w = ew.reshape(IN_C, OUT_H, T).reshape(IN_C, OUT_H * T)
        return jnp.concatenate([w, eb.reshape(IN_C, OUT_H), root], axis=1)

    rhs_n = rhs(ew_n, eb_n, root_n)
    rhs_r = rhs(ew_r, eb_r, root_r)
    bias = jnp.concatenate([b_n, b_r]).reshape(1, 8)

    grid = N // BN
    full = lambda s: pl.BlockSpec(s, lambda j: (0,) * len(s))
    gcat, c16, base = pl.pallas_call(
        _tc_prep_body,
        grid=(grid,),
        in_specs=[
            pl.BlockSpec((BN, IN_C), lambda j: (j, 0)),
            full((1, IN_C)), full((1, IN_C)),
            full((1, IN_C)), full((1, IN_C)),
            full((IN_C, 72)), full((IN_C, 72)),
            full((1, 8)),
        ],
        out_specs=[
            pl.BlockSpec((BN, GW), lambda j: (j, 0)),
            pl.BlockSpec((BN, CW), lambda j: (j, 0)),
            pl.BlockSpec((BN, 8), lambda j: (j, 0)),
        ],
        out_shape=[
            jax.ShapeDtypeStruct((N, GW), jnp.float32),
            jax.ShapeDtypeStruct((N, CW), jnp.float32),
            jax.ShapeDtypeStruct((N, 8), jnp.float32),
        ],
    )(x, ln_g_n.reshape(1, IN_C), ln_b_n.reshape(1, IN_C),
      ln_g_r.reshape(1, IN_C), ln_b_r.reshape(1, IN_C),
      rhs_n, rhs_r, bias)

    mesh = plsc.VectorSubcoreMesh(core_axis_name="c", subcore_axis_name="s",
                                  num_cores=NC, num_subcores=NS)
    partg, partc = pl.kernel(
        _sc_edge_body,
        out_type=[
            jax.ShapeDtypeStruct((NC, NACC, GW), jnp.float32),
            jax.ShapeDtypeStruct((NC, NACC, CW), jnp.float32),
        ],
        mesh=mesh,
        compiler_params=pltpu.CompilerParams(use_tc_tiling_on_sc=False),
        scratch_types=[
            pltpu.VMEM_SHARED((NACC, GW), jnp.float32),
            pltpu.VMEM_SHARED((NACC, CW), jnp.float32),
            pltpu.VMEM((NCHUNK, CH), jnp.int32),
            pltpu.VMEM((NCHUNK, CH), jnp.int32),
            pltpu.VMEM((2, CH, T), jnp.float32),
            pltpu.VMEM((2, CH, T), jnp.float32),
            pltpu.VMEM((2, CH, GW), jnp.float32),
            pltpu.VMEM((2, CH, CW), jnp.float32),
            pltpu.VMEM((ZR, GW), jnp.float32),
            pltpu.VMEM((ZR, CW), jnp.float32),
            pltpu.SemaphoreType.DMA,
            pltpu.SemaphoreType.DMA,
            pltpu.SemaphoreType.DMA,
            pltpu.SemaphoreType.DMA,
        ],
    )(gcat, c16, edge_attr,
      edge_index[0].reshape(NW, NCHUNK, CH),
      edge_index[1].reshape(NW, NCHUNK, CH))

    out = pl.pallas_call(
        _tc_finish_body,
        grid=(grid,),
        in_specs=[
            pl.BlockSpec((NC, BN, GW), lambda j: (0, j, 0)),
            pl.BlockSpec((NC, BN, CW), lambda j: (0, j, 0)),
            pl.BlockSpec((BN, 8), lambda j: (j, 0)),
        ],
        out_specs=pl.BlockSpec((BN, 8), lambda j: (j, 0)),
        out_shape=jax.ShapeDtypeStruct((N, 8), jnp.float32),
    )(partg, partc, base)
    return out
